# baseline (speedup 1.0000x reference)
import jax
import jax.numpy as jnp
from jax.experimental import pallas as pl
from jax.experimental.pallas import tpu as pltpu

N_NODES = 50000
NUM_USERS = 25000
EMB = 64
INTENT = 128
N_EDGES = 800000
BATCH = 4096
LAYERS = 3
E_CHUNK = 4096
N_ECHUNKS = -(-N_EDGES // E_CHUNK)
N_EDGES_PAD = E_CHUNK * N_ECHUNKS
ROW_BLK = 512
N_RBLKS = BATCH // ROW_BLK
SSL_TEMP = 0.2


def _prop_body(emb_ref, src_ref, dst_ref, w_ref, x1_ref, x2_ref, x3_ref,
               a_ref, b_ref, sem):
    layer = pl.program_id(0)
    chunk = pl.program_id(1)
    last = N_ECHUNKS - 1

    @pl.when(chunk == 0)
    def _start_layer():
        @pl.when(layer == 0)
        def _():
            pltpu.make_async_copy(emb_ref, a_ref, sem).start()
            pltpu.make_async_copy(emb_ref, a_ref, sem).wait()

        @pl.when(layer > 0)
        def _():
            a_ref[...] = b_ref[...]

        b_ref[...] = jnp.zeros_like(b_ref)

    def edge(e, _):
        s = src_ref[e]
        d = dst_ref[e]
        wt = w_ref[e]
        row = a_ref[pl.ds(s, 1), :]
        cur = b_ref[pl.ds(d, 1), :]
        b_ref[pl.ds(d, 1), :] = cur + wt * row
        return 0

    jax.lax.fori_loop(0, E_CHUNK, edge, 0)

    @pl.when(chunk == last)
    def _flush():
        for k, out in enumerate((x1_ref, x2_ref, x3_ref)):
            @pl.when(layer == k)
            def _(out=out):
                pltpu.make_async_copy(b_ref, out, sem).start()
                pltpu.make_async_copy(b_ref, out, sem).wait()


def _propagate(all_emb, src, dst, w):
    node = jax.ShapeDtypeStruct((N_NODES, EMB), jnp.float32)
    return pl.pallas_call(
        _prop_body,
        grid=(LAYERS, N_ECHUNKS),
        in_specs=[
            pl.BlockSpec(memory_space=pl.ANY),
            pl.BlockSpec((E_CHUNK,), lambda l, c: (c,), memory_space=pltpu.SMEM),
            pl.BlockSpec((E_CHUNK,), lambda l, c: (c,), memory_space=pltpu.SMEM),
            pl.BlockSpec((E_CHUNK,), lambda l, c: (c,), memory_space=pltpu.SMEM),
        ],
        out_specs=[
            pl.BlockSpec(memory_space=pl.ANY),
            pl.BlockSpec(memory_space=pl.ANY),
            pl.BlockSpec(memory_space=pl.ANY),
        ],
        out_shape=[node, node, node],
        scratch_shapes=[
            pltpu.VMEM((N_NODES, EMB), jnp.float32),
            pltpu.VMEM((N_NODES, EMB), jnp.float32),
            pltpu.SemaphoreType.DMA,
        ],
    )(all_emb, src, dst, w)


def _intent_body(g1_ref, g2_ref, g3_ref, w_ref, n_ref, fin_ref, int_ref):
    g = g1_ref[...] + g2_ref[...] + g3_ref[...]
    w = w_ref[...]
    logits = jnp.dot(g, w, preferred_element_type=jnp.float32)
    m = jnp.max(logits, axis=1, keepdims=True)
    e = jnp.exp(logits - m)
    p = e / jnp.sum(e, axis=1, keepdims=True)
    proj = jax.lax.dot_general(
        p, w, (((1,), (1,)), ((), ())), preferred_element_type=jnp.float32)
    int_ref[...] = proj
    fin_ref[...] = g + proj * n_ref[...]


def _intent(g1, g2, g3, w, noise_half):
    n = g1.shape[0]
    blk = 1000 if n % 1000 == 0 else n
    return pl.pallas_call(
        _intent_body,
        grid=(n // blk,),
        in_specs=[
            pl.BlockSpec((blk, EMB), lambda i: (i, 0)),
            pl.BlockSpec((blk, EMB), lambda i: (i, 0)),
            pl.BlockSpec((blk, EMB), lambda i: (i, 0)),
            pl.BlockSpec((EMB, INTENT), lambda i: (0, 0)),
            pl.BlockSpec((blk, EMB), lambda i: (i, 0)),
        ],
        out_specs=[
            pl.BlockSpec((blk, EMB), lambda i: (i, 0)),
            pl.BlockSpec((blk, EMB), lambda i: (i, 0)),
        ],
        out_shape=[
            jax.ShapeDtypeStruct((n, EMB), jnp.float32),
            jax.ShapeDtypeStruct((n, EMB), jnp.float32),
        ],
    )(g1, g2, g3, w, noise_half)


def _nrm(x):
    return x / (jnp.sqrt(jnp.sum(x * x, axis=1, keepdims=True)) + 1e-8)


def _loss_body(ueb_ref, peb_ref, neb_ref, uef_ref, pef_ref, iub_ref, iib_ref,
               iuf_ref, iif_ref, egu_ref, egp_ref, egn_ref, wu_ref, wi_ref,
               out_ref):
    i = pl.program_id(0)

    @pl.when(i == 0)
    def _():
        out_ref[...] = jnp.zeros_like(out_ref)

    ueb = ueb_ref[...]
    peb = peb_ref[...]
    neb = neb_ref[...]

    ps = jnp.sum(ueb * peb, axis=1)
    ns = jnp.sum(ueb * neb, axis=1)
    z = ps - ns
    bpr = jnp.sum(z - jnp.log1p(jnp.exp(z)))

    reg = (jnp.sum(egu_ref[...] ** 2) + jnp.sum(egp_ref[...] ** 2)
           + jnp.sum(egn_ref[...] ** 2))
    reg_w = jnp.where(
        i == 0, jnp.sum(wu_ref[...] ** 2) + jnp.sum(wi_ref[...] ** 2), 0.0)

    def nce(v1b, v2b, v2f):
        v1 = _nrm(v1b)
        v2 = _nrm(v2f)
        pos = jnp.sum(v1 * _nrm(v2b), axis=1) / SSL_TEMP
        s = jax.lax.dot_general(
            v1, v2, (((1,), (1,)), ((), ())),
            preferred_element_type=jnp.float32) / SSL_TEMP
        m = jnp.max(s, axis=1)
        ttl = m + jnp.log(jnp.sum(jnp.exp(s - m[:, None]), axis=1))
        return jnp.sum(pos - ttl)

    s1 = nce(ueb, ueb, uef_ref[...])
    s2 = nce(peb, peb, pef_ref[...])
    s3 = nce(ueb, peb, pef_ref[...])
    s4 = nce(iub_ref[...], iub_ref[...], iuf_ref[...])
    s5 = nce(iib_ref[...], iib_ref[...], iif_ref[...])

    lane = jax.lax.broadcasted_iota(jnp.int32, (1, 8), 1)
    vals = [bpr, s1, s2, s3, s4, s5, reg, reg_w]
    row = jnp.zeros((1, 8), jnp.float32)
    for k, v in enumerate(vals):
        row = row + jnp.where(lane == k, v, 0.0)
    out_ref[...] = out_ref[...] + row


def _losses(ue, pe, ne, iu, ii, egu, egp, egn, wu, wi):
    blk = lambda i: (i, 0)
    full = lambda i: (0, 0)
    return pl.pallas_call(
        _loss_body,
        grid=(N_RBLKS,),
        in_specs=[
            pl.BlockSpec((ROW_BLK, EMB), blk),
            pl.BlockSpec((ROW_BLK, EMB), blk),
            pl.BlockSpec((ROW_BLK, EMB), blk),
            pl.BlockSpec((BATCH, EMB), full),
            pl.BlockSpec((BATCH, EMB), full),
            pl.BlockSpec((ROW_BLK, EMB), blk),
            pl.BlockSpec((ROW_BLK, EMB), blk),
            pl.BlockSpec((BATCH, EMB), full),
            pl.BlockSpec((BATCH, EMB), full),
            pl.BlockSpec((ROW_BLK, EMB), blk),
            pl.BlockSpec((ROW_BLK, EMB), blk),
            pl.BlockSpec((ROW_BLK, EMB), blk),
            pl.BlockSpec((EMB, INTENT), full),
            pl.BlockSpec((EMB, INTENT), full),
        ],
        out_specs=pl.BlockSpec((1, 8), lambda i: (0, 0)),
        out_shape=jax.ShapeDtypeStruct((1, 8), jnp.float32),
    )(ue, pe, ne, ue, pe, iu, ii, iu, ii, egu, egp, egn, wu, wi)


def kernel(user, positive, negative, user_emb, item_emb, user_intent,
           item_intent, edge_index, edge_weight):
    all_emb = jnp.concatenate([user_emb, item_emb], axis=0)
    pad = N_EDGES_PAD - edge_weight.shape[0]
    src = jnp.pad(edge_index[0], (0, pad))
    dst = jnp.pad(edge_index[1], (0, pad))
    ew = jnp.pad(edge_weight, (0, pad))
    x1, x2, x3 = _propagate(all_emb, src, dst, ew)

    noise = jax.random.normal(jax.random.key(42), (N_NODES, EMB), jnp.float32)
    u_fin, u_int = _intent(x1[:NUM_USERS], x2[:NUM_USERS], x3[:NUM_USERS],
                           user_intent, noise[:NUM_USERS])
    i_fin, i_int = _intent(x1[NUM_USERS:], x2[NUM_USERS:], x3[NUM_USERS:],
                           item_intent, noise[NUM_USERS:])

    ue = u_fin[user]
    pe = i_fin[positive]
    ne = i_fin[negative]
    iu = u_int[user]
    ii = i_int[positive]
    egu = user_emb[user]
    egp = item_emb[positive]
    egn = item_emb[negative]

    p = _losses(ue, pe, ne, iu, ii, egu, egp, egn, user_intent, item_intent)[0]

    bpr_loss = -p[0] / BATCH
    reg_loss = 1e-4 * 0.5 * (p[6] + p[7]) / BATCH
    ssl_loss = 0.2 * (-(p[1] + p[2] + p[3] + p[4] + p[5]) / BATCH)
    return (bpr_loss, reg_loss, ssl_loss)


# speedup vs baseline: 1.9979x; 1.9979x over previous
import jax
import jax.numpy as jnp
from jax.experimental import pallas as pl
from jax.experimental.pallas import tpu as pltpu

N_NODES = 50000
NUM_USERS = 25000
EMB = 64
INTENT = 128
N_EDGES = 800000
BATCH = 4096
LAYERS = 3
E_CHUNK = 4096
N_ECHUNKS = -(-N_EDGES // E_CHUNK)
N_EDGES_PAD = E_CHUNK * N_ECHUNKS
ROW_BLK = 512
N_RBLKS = BATCH // ROW_BLK
SSL_TEMP = 0.2


def _prop_body(emb_ref, src_ref, dst_ref, w_ref, x1_ref, x2_ref, x3_ref,
               a_ref, b_ref, sem):
    layer = pl.program_id(0)
    chunk = pl.program_id(1)
    last = N_ECHUNKS - 1

    @pl.when(chunk == 0)
    def _start_layer():
        @pl.when(layer == 0)
        def _():
            pltpu.make_async_copy(emb_ref, a_ref, sem).start()
            pltpu.make_async_copy(emb_ref, a_ref, sem).wait()

        @pl.when(layer > 0)
        def _():
            a_ref[...] = b_ref[...]

        b_ref[...] = jnp.zeros_like(b_ref)

    def edge(e, _):
        s = src_ref[e]
        d = dst_ref[e]
        wt = w_ref[e]
        row = a_ref[pl.ds(s, 1), :]
        cur = b_ref[pl.ds(d, 1), :]
        b_ref[pl.ds(d, 1), :] = cur + wt * row
        return 0

    jax.lax.fori_loop(0, E_CHUNK, edge, 0, unroll=8)

    @pl.when(chunk == last)
    def _flush():
        for k, out in enumerate((x1_ref, x2_ref, x3_ref)):
            @pl.when(layer == k)
            def _(out=out):
                pltpu.make_async_copy(b_ref, out, sem).start()
                pltpu.make_async_copy(b_ref, out, sem).wait()


def _propagate(all_emb, src, dst, w):
    node = jax.ShapeDtypeStruct((N_NODES, EMB), jnp.float32)
    return pl.pallas_call(
        _prop_body,
        grid=(LAYERS, N_ECHUNKS),
        in_specs=[
            pl.BlockSpec(memory_space=pl.ANY),
            pl.BlockSpec((E_CHUNK,), lambda l, c: (c,), memory_space=pltpu.SMEM),
            pl.BlockSpec((E_CHUNK,), lambda l, c: (c,), memory_space=pltpu.SMEM),
            pl.BlockSpec((E_CHUNK,), lambda l, c: (c,), memory_space=pltpu.SMEM),
        ],
        out_specs=[
            pl.BlockSpec(memory_space=pl.ANY),
            pl.BlockSpec(memory_space=pl.ANY),
            pl.BlockSpec(memory_space=pl.ANY),
        ],
        out_shape=[node, node, node],
        scratch_shapes=[
            pltpu.VMEM((N_NODES, EMB), jnp.float32),
            pltpu.VMEM((N_NODES, EMB), jnp.float32),
            pltpu.SemaphoreType.DMA,
        ],
    )(all_emb, src, dst, w)


def _intent_body(g1_ref, g2_ref, g3_ref, w_ref, n_ref, fin_ref, int_ref):
    g = g1_ref[...] + g2_ref[...] + g3_ref[...]
    w = w_ref[...]
    logits = jnp.dot(g, w, preferred_element_type=jnp.float32)
    m = jnp.max(logits, axis=1, keepdims=True)
    e = jnp.exp(logits - m)
    p = e / jnp.sum(e, axis=1, keepdims=True)
    proj = jax.lax.dot_general(
        p, w, (((1,), (1,)), ((), ())), preferred_element_type=jnp.float32)
    int_ref[...] = proj
    fin_ref[...] = g + proj * n_ref[...]


def _intent(g1, g2, g3, w, noise_half):
    n = g1.shape[0]
    blk = 1000 if n % 1000 == 0 else n
    return pl.pallas_call(
        _intent_body,
        grid=(n // blk,),
        in_specs=[
            pl.BlockSpec((blk, EMB), lambda i: (i, 0)),
            pl.BlockSpec((blk, EMB), lambda i: (i, 0)),
            pl.BlockSpec((blk, EMB), lambda i: (i, 0)),
            pl.BlockSpec((EMB, INTENT), lambda i: (0, 0)),
            pl.BlockSpec((blk, EMB), lambda i: (i, 0)),
        ],
        out_specs=[
            pl.BlockSpec((blk, EMB), lambda i: (i, 0)),
            pl.BlockSpec((blk, EMB), lambda i: (i, 0)),
        ],
        out_shape=[
            jax.ShapeDtypeStruct((n, EMB), jnp.float32),
            jax.ShapeDtypeStruct((n, EMB), jnp.float32),
        ],
    )(g1, g2, g3, w, noise_half)


def _nrm(x):
    return x / (jnp.sqrt(jnp.sum(x * x, axis=1, keepdims=True)) + 1e-8)


def _loss_body(ueb_ref, peb_ref, neb_ref, uef_ref, pef_ref, iub_ref, iib_ref,
               iuf_ref, iif_ref, egu_ref, egp_ref, egn_ref, wu_ref, wi_ref,
               out_ref):
    i = pl.program_id(0)

    @pl.when(i == 0)
    def _():
        out_ref[...] = jnp.zeros_like(out_ref)

    ueb = ueb_ref[...]
    peb = peb_ref[...]
    neb = neb_ref[...]

    ps = jnp.sum(ueb * peb, axis=1)
    ns = jnp.sum(ueb * neb, axis=1)
    z = ps - ns
    bpr = jnp.sum(z - jnp.log1p(jnp.exp(z)))

    reg = (jnp.sum(egu_ref[...] ** 2) + jnp.sum(egp_ref[...] ** 2)
           + jnp.sum(egn_ref[...] ** 2))
    reg_w = jnp.where(
        i == 0, jnp.sum(wu_ref[...] ** 2) + jnp.sum(wi_ref[...] ** 2), 0.0)

    def nce(v1b, v2b, v2f):
        v1 = _nrm(v1b)
        v2 = _nrm(v2f)
        pos = jnp.sum(v1 * _nrm(v2b), axis=1) / SSL_TEMP
        s = jax.lax.dot_general(
            v1, v2, (((1,), (1,)), ((), ())),
            preferred_element_type=jnp.float32) / SSL_TEMP
        m = jnp.max(s, axis=1)
        ttl = m + jnp.log(jnp.sum(jnp.exp(s - m[:, None]), axis=1))
        return jnp.sum(pos - ttl)

    s1 = nce(ueb, ueb, uef_ref[...])
    s2 = nce(peb, peb, pef_ref[...])
    s3 = nce(ueb, peb, pef_ref[...])
    s4 = nce(iub_ref[...], iub_ref[...], iuf_ref[...])
    s5 = nce(iib_ref[...], iib_ref[...], iif_ref[...])

    lane = jax.lax.broadcasted_iota(jnp.int32, (1, 8), 1)
    vals = [bpr, s1, s2, s3, s4, s5, reg, reg_w]
    row = jnp.zeros((1, 8), jnp.float32)
    for k, v in enumerate(vals):
        row = row + jnp.where(lane == k, v, 0.0)
    out_ref[...] = out_ref[...] + row


def _losses(ue, pe, ne, iu, ii, egu, egp, egn, wu, wi):
    blk = lambda i: (i, 0)
    full = lambda i: (0, 0)
    return pl.pallas_call(
        _loss_body,
        grid=(N_RBLKS,),
        in_specs=[
            pl.BlockSpec((ROW_BLK, EMB), blk),
            pl.BlockSpec((ROW_BLK, EMB), blk),
            pl.BlockSpec((ROW_BLK, EMB), blk),
            pl.BlockSpec((BATCH, EMB), full),
            pl.BlockSpec((BATCH, EMB), full),
            pl.BlockSpec((ROW_BLK, EMB), blk),
            pl.BlockSpec((ROW_BLK, EMB), blk),
            pl.BlockSpec((BATCH, EMB), full),
            pl.BlockSpec((BATCH, EMB), full),
            pl.BlockSpec((ROW_BLK, EMB), blk),
            pl.BlockSpec((ROW_BLK, EMB), blk),
            pl.BlockSpec((ROW_BLK, EMB), blk),
            pl.BlockSpec((EMB, INTENT), full),
            pl.BlockSpec((EMB, INTENT), full),
        ],
        out_specs=pl.BlockSpec((1, 8), lambda i: (0, 0)),
        out_shape=jax.ShapeDtypeStruct((1, 8), jnp.float32),
    )(ue, pe, ne, ue, pe, iu, ii, iu, ii, egu, egp, egn, wu, wi)


def kernel(user, positive, negative, user_emb, item_emb, user_intent,
           item_intent, edge_index, edge_weight):
    all_emb = jnp.concatenate([user_emb, item_emb], axis=0)
    pad = N_EDGES_PAD - edge_weight.shape[0]
    src = jnp.pad(edge_index[0], (0, pad))
    dst = jnp.pad(edge_index[1], (0, pad))
    ew = jnp.pad(edge_weight, (0, pad))
    x1, x2, x3 = _propagate(all_emb, src, dst, ew)

    noise = jax.random.normal(jax.random.key(42), (N_NODES, EMB), jnp.float32)
    u_fin, u_int = _intent(x1[:NUM_USERS], x2[:NUM_USERS], x3[:NUM_USERS],
                           user_intent, noise[:NUM_USERS])
    i_fin, i_int = _intent(x1[NUM_USERS:], x2[NUM_USERS:], x3[NUM_USERS:],
                           item_intent, noise[NUM_USERS:])

    ue = u_fin[user]
    pe = i_fin[positive]
    ne = i_fin[negative]
    iu = u_int[user]
    ii = i_int[positive]
    egu = user_emb[user]
    egp = item_emb[positive]
    egn = item_emb[negative]

    p = _losses(ue, pe, ne, iu, ii, egu, egp, egn, user_intent, item_intent)[0]

    bpr_loss = -p[0] / BATCH
    reg_loss = 1e-4 * 0.5 * (p[6] + p[7]) / BATCH
    ssl_loss = 0.2 * (-(p[1] + p[2] + p[3] + p[4] + p[5]) / BATCH)
    return (bpr_loss, reg_loss, ssl_loss)
